# baseline (speedup 1.0000x reference)
import functools

import jax
import jax.numpy as jnp
from jax import lax
from jax.experimental import pallas as pl
from jax.experimental.pallas import tpu as pltpu
from jax.experimental.pallas import tpu_sc as plsc

N_U = 50000
N_M = 10000
E = 500000
L = 100000
D = 128
H = 128

NC = 2
NS = 16
NW = NC * NS

EPW = 16384
EPAD = EPW * NW
NSLOT = 4

NUP = 50048
NMP = 10240

KD = 256
DBLK = 13
LPW = KD * DBLK
LPAD = LPW * NW
LPW2 = LPW + KD

_MESH = plsc.VectorSubcoreMesh(core_axis_name="c", subcore_axis_name="s")


def _zero_acc_slice(zbuf, acc, start, znum, kmax):
    off = 0
    while off < znum:
        n = min(kmax, znum - off)
        pltpu.sync_copy(zbuf.at[pl.ds(0, n)], acc.at[pl.ds(start + off, n)])
        off += n


@functools.cache
def _make_agg(n_src: int, n_dst_p: int, w: int, nch: int, ku: int,
              with_counts: bool):
    Z = n_dst_p // NS
    nbu = EPW // ku
    assert nbu % 2 == 0
    epw2 = EPW + 2 * ku
    kc = ku // 2
    zr = max(512, ku // 2)
    nout = nch + 1 if with_counts else nch
    ixc = epw2 // 4
    zc = max(d for d in range(8, min(zr, Z) + 1, 8) if Z % d == 0)
    assert Z % zc == 0 and zc % 8 == 0 and zc <= zr

    def body(tables, gidx, sidx, zo, out, gv, sv, ba, bb, zbuf, acc,
             ga, gb, sa, sb):
        c = lax.axis_index("c")
        s = lax.axis_index("s")

        def ldix(k, _):
            pltpu.sync_copy(gidx.at[c, s, pl.ds(k * ixc, ixc)],
                            gv.at[pl.ds(k * ixc, ixc)])
            pltpu.sync_copy(sidx.at[c, s, pl.ds(k * ixc, ixc)],
                            sv.at[pl.ds(k * ixc, ixc)])
            return 0

        lax.fori_loop(0, 4, ldix, 0)
        pltpu.sync_copy(zo.at[0, pl.ds(0, zr)], zbuf)

        def gather(cc, blk, buf, sem):
            pltpu.async_copy(tables.at[cc].at[gv.at[pl.ds(blk * ku, ku)]],
                             buf, sem)

        def scat(blk, buf, sem):
            pltpu.async_copy(buf, acc.at[sv.at[pl.ds(blk * ku, ku)]],
                             sem, add=True)

        def drain(cc, buf, sem):
            pltpu.make_async_copy(tables.at[cc].at[pl.ds(0, ku)], buf,
                                  sem).wait()

        for cc in range(nout):
            def zslice(k, _):
                pltpu.sync_copy(zbuf.at[pl.ds(0, zc)],
                                acc.at[pl.ds(s * Z + k * zc, zc)])
                return 0

            lax.fori_loop(0, Z // zc, zslice, 0)
            plsc.subcore_barrier()
            if cc < nch:
                gather(cc, 0, ba, ga)
                gather(cc, 1, bb, gb)

                def pair(h, _):
                    b0 = h * 2
                    drain(cc, ba, ga)
                    scat(b0, ba, sa)
                    drain(cc, bb, gb)
                    scat(b0 + 1, bb, sb)
                    drain(cc, ba, sa)
                    gather(cc, b0 + 2, ba, ga)
                    drain(cc, bb, sb)
                    gather(cc, b0 + 3, bb, gb)
                    return 0

                lax.fori_loop(0, nbu // 2, pair, 0)
                drain(cc, ba, ga)
                drain(cc, bb, gb)
            else:
                pltpu.sync_copy(zo.at[1, pl.ds(0, kc)],
                                zbuf.at[pl.ds(0, kc)])

                def cblk(b, _):
                    pltpu.sync_copy(zbuf.at[pl.ds(0, kc)],
                                    acc.at[sv.at[pl.ds(b * kc, kc)]],
                                    add=True)
                    return 0

                lax.fori_loop(0, EPW // kc, cblk, 0)
            plsc.subcore_barrier()

            def cpout(k, _):
                pltpu.sync_copy(
                    acc.at[pl.ds(s * Z + k * zc, zc)],
                    out.at[c, cc, pl.ds(s * Z + k * zc, zc), :])
                return 0

            lax.fori_loop(0, Z // zc, cpout, 0)
            plsc.subcore_barrier()

    return pl.kernel(
        body,
        out_type=jax.ShapeDtypeStruct((NC, nout, n_dst_p, w), jnp.float32),
        mesh=_MESH,
        scratch_types=[
            pltpu.VMEM((epw2,), jnp.int32),
            pltpu.VMEM((epw2,), jnp.int32),
            pltpu.VMEM((ku, w), jnp.float32),
            pltpu.VMEM((ku, w), jnp.float32),
            pltpu.VMEM((zr, w), jnp.float32),
            pltpu.VMEM_SHARED((n_dst_p, w), jnp.float32),
            pltpu.SemaphoreType.DMA,
            pltpu.SemaphoreType.DMA,
            pltpu.SemaphoreType.DMA,
            pltpu.SemaphoreType.DMA,
        ],
        compiler_params=pltpu.CompilerParams(use_tc_tiling_on_sc=False),
    )


@functools.cache
def _make_label_gather():
    def body(zu, zm, iu_h, im_h, ou, om, iu, im, ba, bb, ga, gb, sa, sb):
        c = lax.axis_index("c")
        s = lax.axis_index("s")
        wid = c * NS + s
        pltpu.sync_copy(iu_h.at[c, s], iu)
        pltpu.sync_copy(im_h.at[c, s], im)
        base = wid * LPW

        def gath(tab, idx, blk, buf, sem):
            pltpu.async_copy(tab.at[idx.at[pl.ds(blk * KD, KD)]], buf, sem)

        def write(o, blk, buf, sem):
            pltpu.async_copy(buf, o.at[pl.ds(base + blk * KD, KD), :], sem)

        def drain(buf, sem):
            pltpu.make_async_copy(zu.at[pl.ds(0, KD)], buf, sem).wait()

        gath(zu, iu, 0, ba, ga)
        gath(zm, im, 0, bb, gb)

        def blkloop(h, _):
            drain(ba, ga)
            write(ou, h, ba, sa)
            drain(bb, gb)
            write(om, h, bb, sb)
            drain(ba, sa)
            gath(zu, iu, h + 1, ba, ga)
            drain(bb, sb)
            gath(zm, im, h + 1, bb, gb)
            return 0

        lax.fori_loop(0, DBLK, blkloop, 0)
        drain(ba, ga)
        drain(bb, gb)

    return pl.kernel(
        body,
        out_type=[
            jax.ShapeDtypeStruct((LPAD, 128), jnp.float32),
            jax.ShapeDtypeStruct((LPAD, 128), jnp.float32),
        ],
        mesh=_MESH,
        scratch_types=[
            pltpu.VMEM((LPW2,), jnp.int32),
            pltpu.VMEM((LPW2,), jnp.int32),
            pltpu.VMEM((KD, 128), jnp.float32),
            pltpu.VMEM((KD, 128), jnp.float32),
            pltpu.SemaphoreType.DMA,
            pltpu.SemaphoreType.DMA,
            pltpu.SemaphoreType.DMA,
            pltpu.SemaphoreType.DMA,
        ],
    )


_RB = 512


def _make_combine1_body(w, nch, wout):
    nco = 128 // wout

    def body(pref, xref, wl, wr, bl, href, hcref, invref):
        p = pref[...]
        agg = jnp.concatenate([p[0, cc] + p[1, cc] for cc in range(nch)],
                              axis=1)
        cnt = p[0, nch, :, 0:1] + p[1, nch, :, 0:1]
        inv = 1.0 / jnp.maximum(cnt, 1.0)
        h = (jnp.dot(agg * inv, wl[...], preferred_element_type=jnp.float32)
             + jnp.dot(xref[...], wr[...], preferred_element_type=jnp.float32)
             + bl[...])
        h = jnp.maximum(h, 0.0)
        href[...] = h
        invref[...] = inv
        for cc in range(nco):
            hcref[cc] = h[:, wout * cc:wout * (cc + 1)]
    return body


def _make_combine2_body(w, nch):
    def body(pref, xref, invref, wl, wr, bl, zref):
        p = pref[...]
        agg = jnp.concatenate([p[0, cc] + p[1, cc] for cc in range(nch)],
                              axis=1)
        zref[...] = (jnp.dot(agg * invref[...], wl[...],
                             preferred_element_type=jnp.float32)
                     + jnp.dot(xref[...], wr[...],
                               preferred_element_type=jnp.float32)
                     + bl[...])
    return body


def _wspec():
    return pl.BlockSpec((128, 128), lambda r: (0, 0))


def _bspec():
    return pl.BlockSpec((1, 128), lambda r: (0, 0))


def _combine1(P, x, Wl, Wr, bl, *, n, npad, w, nch, wout):
    grid = (npad + _RB - 1) // _RB
    nco = 128 // wout
    return pl.pallas_call(
        _make_combine1_body(w, nch, wout),
        grid=(grid,),
        in_specs=[
            pl.BlockSpec((NC, nch + 1, _RB, w), lambda r: (0, 0, r, 0)),
            pl.BlockSpec((_RB, 128), lambda r: (r, 0)),
            _wspec(), _wspec(), _bspec(),
        ],
        out_specs=[
            pl.BlockSpec((_RB, 128), lambda r: (r, 0)),
            pl.BlockSpec((nco, _RB, wout), lambda r: (0, r, 0)),
            pl.BlockSpec((_RB, 1), lambda r: (r, 0)),
        ],
        out_shape=[
            jax.ShapeDtypeStruct((n, 128), jnp.float32),
            jax.ShapeDtypeStruct((nco, n, wout), jnp.float32),
            jax.ShapeDtypeStruct((npad, 1), jnp.float32),
        ],
    )(P, x, Wl, Wr, bl)


def _combine2(P, x, inv, Wl, Wr, bl, *, n, npad, w, nch):
    grid = (npad + _RB - 1) // _RB
    return pl.pallas_call(
        _make_combine2_body(w, nch),
        grid=(grid,),
        in_specs=[
            pl.BlockSpec((NC, nch, _RB, w), lambda r: (0, 0, r, 0)),
            pl.BlockSpec((_RB, 128), lambda r: (r, 0)),
            pl.BlockSpec((_RB, 1), lambda r: (r, 0)),
            _wspec(), _wspec(), _bspec(),
        ],
        out_specs=pl.BlockSpec((_RB, 128), lambda r: (r, 0)),
        out_shape=jax.ShapeDtypeStruct((n, 128), jnp.float32),
    )(P, x, inv, Wl, Wr, bl)


def _decoder_body(zuref, zmref, w1a, w1b, b1, w2, b2, oref):
    h = (jnp.dot(zuref[...], w1a[...], preferred_element_type=jnp.float32)
         + jnp.dot(zmref[...], w1b[...], preferred_element_type=jnp.float32)
         + b1[...])
    h = jnp.maximum(h, 0.0)
    oref[...] = jnp.sum(h * w2[...], axis=1, keepdims=True) + b2[...]


def _decoder(zug, zmg, Wd1, bd1, Wd2, bd2):
    grid = LPAD // _RB
    return pl.pallas_call(
        _decoder_body,
        grid=(grid,),
        in_specs=[
            pl.BlockSpec((_RB, 128), lambda r: (r, 0)),
            pl.BlockSpec((_RB, 128), lambda r: (r, 0)),
            _wspec(), _wspec(), _bspec(), _bspec(),
            pl.BlockSpec((1, 1), lambda r: (0, 0)),
        ],
        out_specs=pl.BlockSpec((_RB, 1), lambda r: (r, 0)),
        out_shape=jax.ShapeDtypeStruct((LPAD, 1), jnp.float32),
    )(zug, zmg, Wd1[:128], Wd1[128:], bd1.reshape(1, 128),
      Wd2.reshape(1, 128), bd2.reshape(1, 1))


def _prep_edges(ix, padval, ku):
    epw2 = EPW + 2 * ku
    body = jnp.concatenate([ix, jnp.full((EPAD - E,), padval, jnp.int32)])
    body = body.reshape(NC, NS, EPW)
    tail = jnp.zeros((NC, NS, epw2 - EPW), jnp.int32)
    return jnp.concatenate([body, tail], axis=2)


def _prep_labels(ix):
    pad = jnp.zeros((LPAD - L,), jnp.int32)
    body = jnp.concatenate([ix, pad]).reshape(NC, NS, LPW)
    tail = jnp.zeros((NC, NS, LPW2 - LPW), jnp.int32)
    return jnp.concatenate([body, tail], axis=2)


def _chunks(x, w):
    nch = 128 // w
    return jnp.stack([x[:, w * cc:w * (cc + 1)] for cc in range(nch)])


def kernel(x_user, x_movie, edge_src, edge_dst, label_src, label_dst,
           Wl1u, bl1u, Wr1u, Wl1m, bl1m, Wr1m,
           Wl2u, bl2u, Wr2u, Wl2m, bl2m, Wr2m,
           Wd1, bd1, Wd2, bd2):
    eg_d_u = _prep_edges(edge_dst, 0, 2048)
    es_u = _prep_edges(edge_src, N_U, 2048)
    eg_s_m = _prep_edges(edge_src, 0, 512)
    es_m = _prep_edges(edge_dst, N_M, 512)

    xm_c = _chunks(x_movie, 8)
    xu_c = _chunks(x_user, 32)

    agg_u1 = _make_agg(N_M, NUP, 8, 16, 2048, True)
    agg_u2 = _make_agg(N_M, NUP, 8, 16, 2048, False)
    agg_m1 = _make_agg(N_U, NMP, 32, 4, 512, True)
    agg_m2 = _make_agg(N_U, NMP, 32, 4, 512, False)

    zo8 = jnp.stack([jnp.zeros((1024, 8), jnp.float32),
                     jnp.ones((1024, 8), jnp.float32)])
    zo32 = jnp.stack([jnp.zeros((512, 32), jnp.float32),
                      jnp.ones((512, 32), jnp.float32)])
    Pu1 = agg_u1(xm_c, eg_d_u, es_u, zo8)
    tok = (Pu1[0, 0, 0, 0] * 0.0).astype(jnp.int32)
    Pm1 = agg_m1(xu_c, eg_s_m, es_m + tok, zo32)

    h_u, hu_c, inv_u = _combine1(
        Pu1, x_user, Wl1u, Wr1u, bl1u.reshape(1, 128), n=N_U, npad=NUP,
        w=8, nch=16, wout=32)
    h_m, hm_c, inv_m = _combine1(
        Pm1, x_movie, Wl1m, Wr1m, bl1m.reshape(1, 128), n=N_M, npad=NMP,
        w=32, nch=4, wout=8)

    tok1 = (Pm1[0, 0, 0, 0] * 0.0).astype(jnp.int32)
    Pu2 = agg_u2(hm_c, eg_d_u, es_u + tok1, zo8)
    tok2 = (Pu2[0, 0, 0, 0] * 0.0).astype(jnp.int32)
    Pm2 = agg_m2(hu_c, eg_s_m, es_m + tok2, zo32)

    z_u = _combine2(Pu2, h_u, inv_u, Wl2u, Wr2u, bl2u.reshape(1, 128),
                    n=N_U, npad=NUP, w=8, nch=16)
    z_m = _combine2(Pm2, h_m, inv_m, Wl2m, Wr2m, bl2m.reshape(1, 128),
                    n=N_M, npad=NMP, w=32, nch=4)

    zug, zmg = _make_label_gather()(z_u, z_m, _prep_labels(label_src),
                                    _prep_labels(label_dst))
    o = _decoder(zug, zmg, Wd1, bd1, Wd2, bd2)
    return o[:L, 0]


# speedup vs baseline: 2.1265x; 2.1265x over previous
import functools

import jax
import jax.numpy as jnp
from jax import lax
from jax.experimental import pallas as pl
from jax.experimental.pallas import tpu as pltpu
from jax.experimental.pallas import tpu_sc as plsc

N_U = 50000
N_M = 10000
E = 500000
L = 100000
D = 128
H = 128

NC = 2
NS = 16
NW = NC * NS

EPW = 16384
EPAD = EPW * NW
NSLOT = 4

NUP = 50048
NMP = 10240

KD = 256
DBLK = 13
LPW = KD * DBLK
LPAD = LPW * NW
LPW2 = LPW + KD

_MESH = plsc.VectorSubcoreMesh(core_axis_name="c", subcore_axis_name="s")


def _zero_acc_slice(zbuf, acc, start, znum, kmax):
    off = 0
    while off < znum:
        n = min(kmax, znum - off)
        pltpu.sync_copy(zbuf.at[pl.ds(0, n)], acc.at[pl.ds(start + off, n)])
        off += n


@functools.cache
def _make_agg(n_src: int, n_dst_p: int, w: int, nch: int, ku: int,
              with_counts: bool):
    Z = n_dst_p // NS
    nbu = EPW // ku
    assert nbu % 2 == 0
    epw2 = EPW + 2 * ku
    zr = max(512, min(1024, ku // 2))
    kc = min(ku // 2, zr)
    nout = nch + 1 if with_counts else nch
    ixc = epw2 // 4
    zc = max(d for d in range(8, min(zr, Z) + 1, 8) if Z % d == 0)
    assert Z % zc == 0 and zc % 8 == 0 and zc <= zr

    def body(tables, gidx, sidx, zo, out, gv, sv, ba, bb, zbuf, acc,
             ga, gb, sa, sb):
        c = lax.axis_index("c")
        s = lax.axis_index("s")

        def ldix(k, _):
            pltpu.sync_copy(gidx.at[c, s, pl.ds(k * ixc, ixc)],
                            gv.at[pl.ds(k * ixc, ixc)])
            pltpu.sync_copy(sidx.at[c, s, pl.ds(k * ixc, ixc)],
                            sv.at[pl.ds(k * ixc, ixc)])
            return 0

        lax.fori_loop(0, 4, ldix, 0)
        pltpu.sync_copy(zo.at[0, pl.ds(0, zr)], zbuf)

        def gather(cc, blk, buf, sem):
            pltpu.async_copy(tables.at[cc].at[gv.at[pl.ds(blk * ku, ku)]],
                             buf, sem)

        def scat(blk, buf, sem):
            pltpu.async_copy(buf, acc.at[sv.at[pl.ds(blk * ku, ku)]],
                             sem, add=True)

        def drain(cc, buf, sem):
            pltpu.make_async_copy(tables.at[cc].at[pl.ds(0, ku)], buf,
                                  sem).wait()

        for cc in range(nout):
            def zslice(k, _):
                pltpu.sync_copy(zbuf.at[pl.ds(0, zc)],
                                acc.at[pl.ds(s * Z + k * zc, zc)])
                return 0

            lax.fori_loop(0, Z // zc, zslice, 0)
            plsc.subcore_barrier()
            if cc < nch:
                def blk(b, _):
                    pltpu.async_copy(
                        tables.at[cc].at[gv.at[pl.ds(b * ku, ku)]],
                        ba, ga)
                    pltpu.make_async_copy(tables.at[cc].at[pl.ds(0, ku)],
                                          ba, ga).wait()
                    pltpu.sync_copy(ba, acc.at[sv.at[pl.ds(b * ku, ku)]],
                                    add=True)
                    return 0

                lax.fori_loop(0, nbu, blk, 0)
            else:
                pltpu.sync_copy(zo.at[1, pl.ds(0, kc)],
                                zbuf.at[pl.ds(0, kc)])

                def cblk(b, _):
                    pltpu.sync_copy(zbuf.at[pl.ds(0, kc)],
                                    acc.at[sv.at[pl.ds(b * kc, kc)]],
                                    add=True)
                    return 0

                lax.fori_loop(0, EPW // kc, cblk, 0)
            plsc.subcore_barrier()

            def cpout(k, _):
                pltpu.sync_copy(
                    acc.at[pl.ds(s * Z + k * zc, zc)],
                    out.at[c, cc, pl.ds(s * Z + k * zc, zc), :])
                return 0

            lax.fori_loop(0, Z // zc, cpout, 0)
            plsc.subcore_barrier()

    return pl.kernel(
        body,
        out_type=jax.ShapeDtypeStruct((NC, nout, n_dst_p, w), jnp.float32),
        mesh=_MESH,
        scratch_types=[
            pltpu.VMEM((epw2,), jnp.int32),
            pltpu.VMEM((epw2,), jnp.int32),
            pltpu.VMEM((ku, w), jnp.float32),
            pltpu.VMEM((ku, w), jnp.float32),
            pltpu.VMEM((zr, w), jnp.float32),
            pltpu.VMEM_SHARED((n_dst_p, w), jnp.float32),
            pltpu.SemaphoreType.DMA,
            pltpu.SemaphoreType.DMA,
            pltpu.SemaphoreType.DMA,
            pltpu.SemaphoreType.DMA,
        ],
        compiler_params=pltpu.CompilerParams(use_tc_tiling_on_sc=False),
    )


@functools.cache
def _make_label_gather():
    def body(zu, zm, iu_h, im_h, ou, om, iu, im, ba, bb, ga, gb, sa, sb):
        c = lax.axis_index("c")
        s = lax.axis_index("s")
        wid = c * NS + s
        pltpu.sync_copy(iu_h.at[c, s], iu)
        pltpu.sync_copy(im_h.at[c, s], im)
        base = wid * LPW

        def gath(tab, idx, blk, buf, sem):
            pltpu.async_copy(tab.at[idx.at[pl.ds(blk * KD, KD)]], buf, sem)

        def write(o, blk, buf, sem):
            pltpu.async_copy(buf, o.at[pl.ds(base + blk * KD, KD), :], sem)

        def drain(buf, sem):
            pltpu.make_async_copy(zu.at[pl.ds(0, KD)], buf, sem).wait()

        gath(zu, iu, 0, ba, ga)
        gath(zm, im, 0, bb, gb)

        def blkloop(h, _):
            drain(ba, ga)
            write(ou, h, ba, sa)
            drain(bb, gb)
            write(om, h, bb, sb)
            drain(ba, sa)
            gath(zu, iu, h + 1, ba, ga)
            drain(bb, sb)
            gath(zm, im, h + 1, bb, gb)
            return 0

        lax.fori_loop(0, DBLK, blkloop, 0)
        drain(ba, ga)
        drain(bb, gb)

    return pl.kernel(
        body,
        out_type=[
            jax.ShapeDtypeStruct((LPAD, 128), jnp.float32),
            jax.ShapeDtypeStruct((LPAD, 128), jnp.float32),
        ],
        mesh=_MESH,
        scratch_types=[
            pltpu.VMEM((LPW2,), jnp.int32),
            pltpu.VMEM((LPW2,), jnp.int32),
            pltpu.VMEM((KD, 128), jnp.float32),
            pltpu.VMEM((KD, 128), jnp.float32),
            pltpu.SemaphoreType.DMA,
            pltpu.SemaphoreType.DMA,
            pltpu.SemaphoreType.DMA,
            pltpu.SemaphoreType.DMA,
        ],
    )


_RB = 512


def _make_combine1_body(w, nch, wout):
    nco = 128 // wout

    def body(pref, xref, wl, wr, bl, href, hcref, invref):
        p = pref[...]
        agg = jnp.concatenate([p[0, cc] + p[1, cc] for cc in range(nch)],
                              axis=1)
        cnt = p[0, nch, :, 0:1] + p[1, nch, :, 0:1]
        inv = 1.0 / jnp.maximum(cnt, 1.0)
        h = (jnp.dot(agg * inv, wl[...], preferred_element_type=jnp.float32)
             + jnp.dot(xref[...], wr[...], preferred_element_type=jnp.float32)
             + bl[...])
        h = jnp.maximum(h, 0.0)
        href[...] = h
        invref[...] = inv
        for cc in range(nco):
            hcref[cc] = h[:, wout * cc:wout * (cc + 1)]
    return body


def _make_combine2_body(w, nch):
    def body(pref, xref, invref, wl, wr, bl, zref):
        p = pref[...]
        agg = jnp.concatenate([p[0, cc] + p[1, cc] for cc in range(nch)],
                              axis=1)
        zref[...] = (jnp.dot(agg * invref[...], wl[...],
                             preferred_element_type=jnp.float32)
                     + jnp.dot(xref[...], wr[...],
                               preferred_element_type=jnp.float32)
                     + bl[...])
    return body


def _wspec():
    return pl.BlockSpec((128, 128), lambda r: (0, 0))


def _bspec():
    return pl.BlockSpec((1, 128), lambda r: (0, 0))


def _combine1(P, x, Wl, Wr, bl, *, n, npad, w, nch, wout):
    grid = (npad + _RB - 1) // _RB
    nco = 128 // wout
    return pl.pallas_call(
        _make_combine1_body(w, nch, wout),
        grid=(grid,),
        in_specs=[
            pl.BlockSpec((NC, nch + 1, _RB, w), lambda r: (0, 0, r, 0)),
            pl.BlockSpec((_RB, 128), lambda r: (r, 0)),
            _wspec(), _wspec(), _bspec(),
        ],
        out_specs=[
            pl.BlockSpec((_RB, 128), lambda r: (r, 0)),
            pl.BlockSpec((nco, _RB, wout), lambda r: (0, r, 0)),
            pl.BlockSpec((_RB, 1), lambda r: (r, 0)),
        ],
        out_shape=[
            jax.ShapeDtypeStruct((n, 128), jnp.float32),
            jax.ShapeDtypeStruct((nco, n, wout), jnp.float32),
            jax.ShapeDtypeStruct((npad, 1), jnp.float32),
        ],
    )(P, x, Wl, Wr, bl)


def _combine2(P, x, inv, Wl, Wr, bl, *, n, npad, w, nch):
    grid = (npad + _RB - 1) // _RB
    return pl.pallas_call(
        _make_combine2_body(w, nch),
        grid=(grid,),
        in_specs=[
            pl.BlockSpec((NC, nch, _RB, w), lambda r: (0, 0, r, 0)),
            pl.BlockSpec((_RB, 128), lambda r: (r, 0)),
            pl.BlockSpec((_RB, 1), lambda r: (r, 0)),
            _wspec(), _wspec(), _bspec(),
        ],
        out_specs=pl.BlockSpec((_RB, 128), lambda r: (r, 0)),
        out_shape=jax.ShapeDtypeStruct((n, 128), jnp.float32),
    )(P, x, inv, Wl, Wr, bl)


def _decoder_body(zuref, zmref, w1a, w1b, b1, w2, b2, oref):
    h = (jnp.dot(zuref[...], w1a[...], preferred_element_type=jnp.float32)
         + jnp.dot(zmref[...], w1b[...], preferred_element_type=jnp.float32)
         + b1[...])
    h = jnp.maximum(h, 0.0)
    oref[...] = jnp.sum(h * w2[...], axis=1, keepdims=True) + b2[...]


def _decoder(zug, zmg, Wd1, bd1, Wd2, bd2):
    grid = LPAD // _RB
    return pl.pallas_call(
        _decoder_body,
        grid=(grid,),
        in_specs=[
            pl.BlockSpec((_RB, 128), lambda r: (r, 0)),
            pl.BlockSpec((_RB, 128), lambda r: (r, 0)),
            _wspec(), _wspec(), _bspec(), _bspec(),
            pl.BlockSpec((1, 1), lambda r: (0, 0)),
        ],
        out_specs=pl.BlockSpec((_RB, 1), lambda r: (r, 0)),
        out_shape=jax.ShapeDtypeStruct((LPAD, 1), jnp.float32),
    )(zug, zmg, Wd1[:128], Wd1[128:], bd1.reshape(1, 128),
      Wd2.reshape(1, 128), bd2.reshape(1, 1))


def _prep_edges(ix, padval, ku):
    epw2 = EPW + 2 * ku
    body = jnp.concatenate([ix, jnp.full((EPAD - E,), padval, jnp.int32)])
    body = body.reshape(NC, NS, EPW)
    tail = jnp.zeros((NC, NS, epw2 - EPW), jnp.int32)
    return jnp.concatenate([body, tail], axis=2)


def _prep_labels(ix):
    pad = jnp.zeros((LPAD - L,), jnp.int32)
    body = jnp.concatenate([ix, pad]).reshape(NC, NS, LPW)
    tail = jnp.zeros((NC, NS, LPW2 - LPW), jnp.int32)
    return jnp.concatenate([body, tail], axis=2)


def _chunks(x, w):
    nch = 128 // w
    return jnp.stack([x[:, w * cc:w * (cc + 1)] for cc in range(nch)])


def kernel(x_user, x_movie, edge_src, edge_dst, label_src, label_dst,
           Wl1u, bl1u, Wr1u, Wl1m, bl1m, Wr1m,
           Wl2u, bl2u, Wr2u, Wl2m, bl2m, Wr2m,
           Wd1, bd1, Wd2, bd2):
    eg_d_u = _prep_edges(edge_dst, 0, 4096)
    es_u = _prep_edges(edge_src, N_U, 4096)
    eg_s_m = _prep_edges(edge_src, 0, 1024)
    es_m = _prep_edges(edge_dst, N_M, 1024)

    xm_c = _chunks(x_movie, 8)
    xu_c = _chunks(x_user, 32)

    agg_u1 = _make_agg(N_M, NUP, 8, 16, 4096, True)
    agg_u2 = _make_agg(N_M, NUP, 8, 16, 4096, False)
    agg_m1 = _make_agg(N_U, NMP, 32, 4, 1024, True)
    agg_m2 = _make_agg(N_U, NMP, 32, 4, 1024, False)

    zo8 = jnp.stack([jnp.zeros((1024, 8), jnp.float32),
                     jnp.ones((1024, 8), jnp.float32)])
    zo32 = jnp.stack([jnp.zeros((512, 32), jnp.float32),
                      jnp.ones((512, 32), jnp.float32)])
    Pu1 = agg_u1(xm_c, eg_d_u, es_u, zo8)
    tok = (Pu1[0, 0, 0, 0] * 0.0).astype(jnp.int32)
    Pm1 = agg_m1(xu_c, eg_s_m, es_m + tok, zo32)

    h_u, hu_c, inv_u = _combine1(
        Pu1, x_user, Wl1u, Wr1u, bl1u.reshape(1, 128), n=N_U, npad=NUP,
        w=8, nch=16, wout=32)
    h_m, hm_c, inv_m = _combine1(
        Pm1, x_movie, Wl1m, Wr1m, bl1m.reshape(1, 128), n=N_M, npad=NMP,
        w=32, nch=4, wout=8)

    tok1 = (Pm1[0, 0, 0, 0] * 0.0).astype(jnp.int32)
    Pu2 = agg_u2(hm_c, eg_d_u, es_u + tok1, zo8)
    tok2 = (Pu2[0, 0, 0, 0] * 0.0).astype(jnp.int32)
    Pm2 = agg_m2(hu_c, eg_s_m, es_m + tok2, zo32)

    z_u = _combine2(Pu2, h_u, inv_u, Wl2u, Wr2u, bl2u.reshape(1, 128),
                    n=N_U, npad=NUP, w=8, nch=16)
    z_m = _combine2(Pm2, h_m, inv_m, Wl2m, Wr2m, bl2m.reshape(1, 128),
                    n=N_M, npad=NMP, w=32, nch=4)

    zug, zmg = _make_label_gather()(z_u, z_m, _prep_labels(label_src),
                                    _prep_labels(label_dst))
    o = _decoder(zug, zmg, Wd1, bd1, Wd2, bd2)
    return o[:L, 0]


# speedup vs baseline: 2.1872x; 1.0285x over previous
import functools

import jax
import jax.numpy as jnp
from jax import lax
from jax.experimental import pallas as pl
from jax.experimental.pallas import tpu as pltpu
from jax.experimental.pallas import tpu_sc as plsc

N_U = 50000
N_M = 10000
E = 500000
L = 100000
D = 128
H = 128

NC = 2
NS = 16
NW = NC * NS

EPW = 16384
EPAD = EPW * NW
NSLOT = 4

NUP = 50048
NMP = 10240

KD = 256
DBLK = 13
LPW = KD * DBLK
LPAD = LPW * NW
LPW2 = LPW + KD

_MESH = plsc.VectorSubcoreMesh(core_axis_name="c", subcore_axis_name="s")


def _zero_acc_slice(zbuf, acc, start, znum, kmax):
    off = 0
    while off < znum:
        n = min(kmax, znum - off)
        pltpu.sync_copy(zbuf.at[pl.ds(0, n)], acc.at[pl.ds(start + off, n)])
        off += n


@functools.cache
def _make_agg(n_src: int, n_dst_p: int, w: int, nch: int, ku: int,
              with_counts: bool):
    Z = n_dst_p // NS
    nbu = EPW // ku
    assert nbu % 2 == 0
    epw2 = EPW + 2 * ku
    zr = max(512, min(1024, ku // 2))
    kc = min(ku // 2, zr)
    nout = nch + 1 if with_counts else nch
    ixc = epw2 // 4
    zc = max(d for d in range(8, min(zr, Z) + 1, 8) if Z % d == 0)
    assert Z % zc == 0 and zc % 8 == 0 and zc <= zr

    def body(tables, gidx, sidx, zo, out, gv, sv, ba, bb, zbuf, acc,
             ga, gb, sa, sb):
        c = lax.axis_index("c")
        s = lax.axis_index("s")

        def ldix(k, _):
            pltpu.sync_copy(gidx.at[c, s, pl.ds(k * ixc, ixc)],
                            gv.at[pl.ds(k * ixc, ixc)])
            pltpu.sync_copy(sidx.at[c, s, pl.ds(k * ixc, ixc)],
                            sv.at[pl.ds(k * ixc, ixc)])
            return 0

        lax.fori_loop(0, 4, ldix, 0)
        pltpu.sync_copy(zo.at[0, pl.ds(0, zr)], zbuf)

        def gather(cc, blk, buf, sem):
            pltpu.async_copy(tables.at[cc].at[gv.at[pl.ds(blk * ku, ku)]],
                             buf, sem)

        def scat(blk, buf, sem):
            pltpu.async_copy(buf, acc.at[sv.at[pl.ds(blk * ku, ku)]],
                             sem, add=True)

        def drain(cc, buf, sem):
            pltpu.make_async_copy(tables.at[cc].at[pl.ds(0, ku)], buf,
                                  sem).wait()

        for cc in range(nout):
            def zslice(k, _):
                pltpu.sync_copy(zbuf.at[pl.ds(0, zc)],
                                acc.at[pl.ds(s * Z + k * zc, zc)])
                return 0

            lax.fori_loop(0, Z // zc, zslice, 0)
            plsc.subcore_barrier()
            if cc < nch:
                def blk(b, _):
                    pltpu.async_copy(
                        tables.at[cc].at[gv.at[pl.ds(b * ku, ku)]],
                        ba, ga)
                    pltpu.make_async_copy(tables.at[cc].at[pl.ds(0, ku)],
                                          ba, ga).wait()
                    pltpu.sync_copy(ba, acc.at[sv.at[pl.ds(b * ku, ku)]],
                                    add=True)
                    return 0

                lax.fori_loop(0, nbu, blk, 0)
            else:
                pltpu.sync_copy(zo.at[1, pl.ds(0, kc)],
                                zbuf.at[pl.ds(0, kc)])

                def cblk(b, _):
                    pltpu.sync_copy(zbuf.at[pl.ds(0, kc)],
                                    acc.at[sv.at[pl.ds(b * kc, kc)]],
                                    add=True)
                    return 0

                lax.fori_loop(0, EPW // kc, cblk, 0)
            plsc.subcore_barrier()

            def cpout(k, _):
                pltpu.sync_copy(
                    acc.at[pl.ds(s * Z + k * zc, zc)],
                    out.at[c, cc, pl.ds(s * Z + k * zc, zc), :])
                return 0

            lax.fori_loop(0, Z // zc, cpout, 0)
            plsc.subcore_barrier()

    return pl.kernel(
        body,
        out_type=jax.ShapeDtypeStruct((NC, nout, n_dst_p, w), jnp.float32),
        mesh=_MESH,
        scratch_types=[
            pltpu.VMEM((epw2,), jnp.int32),
            pltpu.VMEM((epw2,), jnp.int32),
            pltpu.VMEM((ku, w), jnp.float32),
            pltpu.VMEM((ku, w), jnp.float32),
            pltpu.VMEM((zr, w), jnp.float32),
            pltpu.VMEM_SHARED((n_dst_p, w), jnp.float32),
            pltpu.SemaphoreType.DMA,
            pltpu.SemaphoreType.DMA,
            pltpu.SemaphoreType.DMA,
            pltpu.SemaphoreType.DMA,
        ],
        compiler_params=pltpu.CompilerParams(use_tc_tiling_on_sc=False),
    )


@functools.cache
def _make_label_gather():
    def body(zu, zm, iu_h, im_h, ou, om, iu, im, ba, bb, ga, gb, sa, sb):
        c = lax.axis_index("c")
        s = lax.axis_index("s")
        wid = c * NS + s
        pltpu.sync_copy(iu_h.at[c, s], iu)
        pltpu.sync_copy(im_h.at[c, s], im)
        base = wid * LPW

        def gath(tab, idx, blk, buf, sem):
            pltpu.async_copy(tab.at[idx.at[pl.ds(blk * KD, KD)]], buf, sem)

        def write(o, blk, buf, sem):
            pltpu.async_copy(buf, o.at[pl.ds(base + blk * KD, KD), :], sem)

        def drain(buf, sem):
            pltpu.make_async_copy(zu.at[pl.ds(0, KD)], buf, sem).wait()

        gath(zu, iu, 0, ba, ga)
        gath(zm, im, 0, bb, gb)

        def blkloop(h, _):
            drain(ba, ga)
            write(ou, h, ba, sa)
            drain(bb, gb)
            write(om, h, bb, sb)
            drain(ba, sa)
            gath(zu, iu, h + 1, ba, ga)
            drain(bb, sb)
            gath(zm, im, h + 1, bb, gb)
            return 0

        lax.fori_loop(0, DBLK, blkloop, 0)
        drain(ba, ga)
        drain(bb, gb)

    return pl.kernel(
        body,
        out_type=[
            jax.ShapeDtypeStruct((LPAD, 128), jnp.float32),
            jax.ShapeDtypeStruct((LPAD, 128), jnp.float32),
        ],
        mesh=_MESH,
        scratch_types=[
            pltpu.VMEM((LPW2,), jnp.int32),
            pltpu.VMEM((LPW2,), jnp.int32),
            pltpu.VMEM((KD, 128), jnp.float32),
            pltpu.VMEM((KD, 128), jnp.float32),
            pltpu.SemaphoreType.DMA,
            pltpu.SemaphoreType.DMA,
            pltpu.SemaphoreType.DMA,
            pltpu.SemaphoreType.DMA,
        ],
    )


_RB = 512


def _make_combine1_body(w, nch, wout):
    nco = 128 // wout

    def body(pref, xref, wl, wr, bl, href, hcref, invref):
        p = pref[...]
        agg = jnp.concatenate([p[0, cc] + p[1, cc] for cc in range(nch)],
                              axis=1)
        cnt = p[0, nch, :, 0:1] + p[1, nch, :, 0:1]
        inv = 1.0 / jnp.maximum(cnt, 1.0)
        h = (jnp.dot(agg * inv, wl[...], preferred_element_type=jnp.float32)
             + jnp.dot(xref[...], wr[...], preferred_element_type=jnp.float32)
             + bl[...])
        h = jnp.maximum(h, 0.0)
        href[...] = h
        invref[...] = inv
        for cc in range(nco):
            hcref[cc] = h[:, wout * cc:wout * (cc + 1)]
    return body


def _make_combine2_body(w, nch):
    def body(pref, xref, invref, wl, wr, bl, zref):
        p = pref[...]
        agg = jnp.concatenate([p[0, cc] + p[1, cc] for cc in range(nch)],
                              axis=1)
        zref[...] = (jnp.dot(agg * invref[...], wl[...],
                             preferred_element_type=jnp.float32)
                     + jnp.dot(xref[...], wr[...],
                               preferred_element_type=jnp.float32)
                     + bl[...])
    return body


def _wspec():
    return pl.BlockSpec((128, 128), lambda r: (0, 0))


def _bspec():
    return pl.BlockSpec((1, 128), lambda r: (0, 0))


def _combine1(P, x, Wl, Wr, bl, *, n, npad, w, nch, wout):
    grid = (npad + _RB - 1) // _RB
    nco = 128 // wout
    return pl.pallas_call(
        _make_combine1_body(w, nch, wout),
        grid=(grid,),
        in_specs=[
            pl.BlockSpec((NC, nch + 1, _RB, w), lambda r: (0, 0, r, 0)),
            pl.BlockSpec((_RB, 128), lambda r: (r, 0)),
            _wspec(), _wspec(), _bspec(),
        ],
        out_specs=[
            pl.BlockSpec((_RB, 128), lambda r: (r, 0)),
            pl.BlockSpec((nco, _RB, wout), lambda r: (0, r, 0)),
            pl.BlockSpec((_RB, 1), lambda r: (r, 0)),
        ],
        out_shape=[
            jax.ShapeDtypeStruct((n, 128), jnp.float32),
            jax.ShapeDtypeStruct((nco, n, wout), jnp.float32),
            jax.ShapeDtypeStruct((npad, 1), jnp.float32),
        ],
    )(P, x, Wl, Wr, bl)


def _combine2(P, x, inv, Wl, Wr, bl, *, n, npad, w, nch):
    grid = (npad + _RB - 1) // _RB
    return pl.pallas_call(
        _make_combine2_body(w, nch),
        grid=(grid,),
        in_specs=[
            pl.BlockSpec((NC, nch, _RB, w), lambda r: (0, 0, r, 0)),
            pl.BlockSpec((_RB, 128), lambda r: (r, 0)),
            pl.BlockSpec((_RB, 1), lambda r: (r, 0)),
            _wspec(), _wspec(), _bspec(),
        ],
        out_specs=pl.BlockSpec((_RB, 128), lambda r: (r, 0)),
        out_shape=jax.ShapeDtypeStruct((n, 128), jnp.float32),
    )(P, x, inv, Wl, Wr, bl)


def _decoder_body(zuref, zmref, w1a, w1b, b1, w2, b2, oref):
    h = (jnp.dot(zuref[...], w1a[...], preferred_element_type=jnp.float32)
         + jnp.dot(zmref[...], w1b[...], preferred_element_type=jnp.float32)
         + b1[...])
    h = jnp.maximum(h, 0.0)
    oref[...] = jnp.sum(h * w2[...], axis=1, keepdims=True) + b2[...]


def _decoder(zug, zmg, Wd1, bd1, Wd2, bd2):
    grid = LPAD // _RB
    return pl.pallas_call(
        _decoder_body,
        grid=(grid,),
        in_specs=[
            pl.BlockSpec((_RB, 128), lambda r: (r, 0)),
            pl.BlockSpec((_RB, 128), lambda r: (r, 0)),
            _wspec(), _wspec(), _bspec(), _bspec(),
            pl.BlockSpec((1, 1), lambda r: (0, 0)),
        ],
        out_specs=pl.BlockSpec((_RB, 1), lambda r: (r, 0)),
        out_shape=jax.ShapeDtypeStruct((LPAD, 1), jnp.float32),
    )(zug, zmg, Wd1[:128], Wd1[128:], bd1.reshape(1, 128),
      Wd2.reshape(1, 128), bd2.reshape(1, 1))


def _prep_edges(ix, padval, ku):
    epw2 = EPW + 2 * ku
    body = jnp.concatenate([ix, jnp.full((EPAD - E,), padval, jnp.int32)])
    body = body.reshape(NC, NS, EPW)
    tail = jnp.zeros((NC, NS, epw2 - EPW), jnp.int32)
    return jnp.concatenate([body, tail], axis=2)


def _prep_labels(ix):
    pad = jnp.zeros((LPAD - L,), jnp.int32)
    body = jnp.concatenate([ix, pad]).reshape(NC, NS, LPW)
    tail = jnp.zeros((NC, NS, LPW2 - LPW), jnp.int32)
    return jnp.concatenate([body, tail], axis=2)


def _chunks(x, w):
    nch = 128 // w
    return jnp.stack([x[:, w * cc:w * (cc + 1)] for cc in range(nch)])


def kernel(x_user, x_movie, edge_src, edge_dst, label_src, label_dst,
           Wl1u, bl1u, Wr1u, Wl1m, bl1m, Wr1m,
           Wl2u, bl2u, Wr2u, Wl2m, bl2m, Wr2m,
           Wd1, bd1, Wd2, bd2):
    eg_d_u = _prep_edges(edge_dst, 0, 1024)
    es_u = _prep_edges(edge_src, N_U, 1024)
    eg_s_m = _prep_edges(edge_src, 0, 1024)
    es_m = _prep_edges(edge_dst, N_M, 1024)

    xm_c = _chunks(x_movie, 8)
    xu_c = _chunks(x_user, 32)

    agg_u1 = _make_agg(N_M, NUP, 8, 16, 1024, True)
    agg_u2 = _make_agg(N_M, NUP, 8, 16, 1024, False)
    agg_m1 = _make_agg(N_U, NMP, 32, 4, 1024, True)
    agg_m2 = _make_agg(N_U, NMP, 32, 4, 1024, False)

    zo8 = jnp.stack([jnp.zeros((1024, 8), jnp.float32),
                     jnp.ones((1024, 8), jnp.float32)])
    zo32 = jnp.stack([jnp.zeros((512, 32), jnp.float32),
                      jnp.ones((512, 32), jnp.float32)])
    Pu1 = agg_u1(xm_c, eg_d_u, es_u, zo8)
    tok = (Pu1[0, 0, 0, 0] * 0.0).astype(jnp.int32)
    Pm1 = agg_m1(xu_c, eg_s_m, es_m + tok, zo32)

    h_u, hu_c, inv_u = _combine1(
        Pu1, x_user, Wl1u, Wr1u, bl1u.reshape(1, 128), n=N_U, npad=NUP,
        w=8, nch=16, wout=32)
    h_m, hm_c, inv_m = _combine1(
        Pm1, x_movie, Wl1m, Wr1m, bl1m.reshape(1, 128), n=N_M, npad=NMP,
        w=32, nch=4, wout=8)

    tok1 = (Pm1[0, 0, 0, 0] * 0.0).astype(jnp.int32)
    Pu2 = agg_u2(hm_c, eg_d_u, es_u + tok1, zo8)
    tok2 = (Pu2[0, 0, 0, 0] * 0.0).astype(jnp.int32)
    Pm2 = agg_m2(hu_c, eg_s_m, es_m + tok2, zo32)

    z_u = _combine2(Pu2, h_u, inv_u, Wl2u, Wr2u, bl2u.reshape(1, 128),
                    n=N_U, npad=NUP, w=8, nch=16)
    z_m = _combine2(Pm2, h_m, inv_m, Wl2m, Wr2m, bl2m.reshape(1, 128),
                    n=N_M, npad=NMP, w=32, nch=4)

    zug, zmg = _make_label_gather()(z_u, z_m, _prep_labels(label_src),
                                    _prep_labels(label_dst))
    o = _decoder(zug, zmg, Wd1, bd1, Wd2, bd2)
    return o[:L, 0]


# speedup vs baseline: 2.1874x; 1.0001x over previous
import functools

import jax
import jax.numpy as jnp
from jax import lax
from jax.experimental import pallas as pl
from jax.experimental.pallas import tpu as pltpu
from jax.experimental.pallas import tpu_sc as plsc

N_U = 50000
N_M = 10000
E = 500000
L = 100000
D = 128
H = 128

NC = 2
NS = 16
NW = NC * NS

EPW = 16384
EPAD = EPW * NW
NSLOT = 4

NUP = 50048
NMP = 10240

KD = 256
DBLK = 13
LPW = KD * DBLK
LPAD = LPW * NW
LPW2 = LPW + KD

_MESH = plsc.VectorSubcoreMesh(core_axis_name="c", subcore_axis_name="s")


def _zero_acc_slice(zbuf, acc, start, znum, kmax):
    off = 0
    while off < znum:
        n = min(kmax, znum - off)
        pltpu.sync_copy(zbuf.at[pl.ds(0, n)], acc.at[pl.ds(start + off, n)])
        off += n


@functools.cache
def _make_agg(n_src: int, n_dst_p: int, w: int, nch: int, ku: int,
              with_counts: bool):
    Z = n_dst_p // NS
    nbu = EPW // ku
    assert nbu % 2 == 0
    epw2 = EPW + 2 * ku
    zr = max(512, min(1024, ku // 2))
    kc = min(ku // 2, zr)
    nout = nch + 1 if with_counts else nch
    ixc = epw2 // 4
    zc = max(d for d in range(8, min(zr, Z) + 1, 8) if Z % d == 0)
    assert Z % zc == 0 and zc % 8 == 0 and zc <= zr

    def body(tables, gidx, sidx, zo, out, gv, sv, ba, bb, zbuf, acc,
             ga, gb, sa, sb):
        c = lax.axis_index("c")
        s = lax.axis_index("s")

        def ldix(k, _):
            pltpu.sync_copy(gidx.at[c, s, pl.ds(k * ixc, ixc)],
                            gv.at[pl.ds(k * ixc, ixc)])
            pltpu.sync_copy(sidx.at[c, s, pl.ds(k * ixc, ixc)],
                            sv.at[pl.ds(k * ixc, ixc)])
            return 0

        lax.fori_loop(0, 4, ldix, 0)
        pltpu.sync_copy(zo.at[0, pl.ds(0, zr)], zbuf)

        def gather(cc, blk, buf, sem):
            pltpu.async_copy(tables.at[cc].at[gv.at[pl.ds(blk * ku, ku)]],
                             buf, sem)

        def scat(blk, buf, sem):
            pltpu.async_copy(buf, acc.at[sv.at[pl.ds(blk * ku, ku)]],
                             sem, add=True)

        def drain(cc, buf, sem):
            pltpu.make_async_copy(tables.at[cc].at[pl.ds(0, ku)], buf,
                                  sem).wait()

        for cc in range(nout):
            def zslice(k, _):
                pltpu.sync_copy(zbuf.at[pl.ds(0, zc)],
                                acc.at[pl.ds(s * Z + k * zc, zc)])
                return 0

            lax.fori_loop(0, Z // zc, zslice, 0)
            plsc.subcore_barrier()
            if cc < nch:
                def blk(b, _):
                    pltpu.async_copy(
                        tables.at[cc].at[gv.at[pl.ds(b * ku, ku)]],
                        ba, ga).wait()
                    pltpu.sync_copy(ba, acc.at[sv.at[pl.ds(b * ku, ku)]],
                                    add=True)
                    return 0

                lax.fori_loop(0, nbu, blk, 0)
            else:
                pltpu.sync_copy(zo.at[1, pl.ds(0, kc)],
                                zbuf.at[pl.ds(0, kc)])

                def cblk(b, _):
                    pltpu.sync_copy(zbuf.at[pl.ds(0, kc)],
                                    acc.at[sv.at[pl.ds(b * kc, kc)]],
                                    add=True)
                    return 0

                lax.fori_loop(0, EPW // kc, cblk, 0)
            plsc.subcore_barrier()

            def cpout(k, _):
                pltpu.sync_copy(
                    acc.at[pl.ds(s * Z + k * zc, zc)],
                    out.at[c, cc, pl.ds(s * Z + k * zc, zc), :])
                return 0

            lax.fori_loop(0, Z // zc, cpout, 0)
            plsc.subcore_barrier()

    return pl.kernel(
        body,
        out_type=jax.ShapeDtypeStruct((NC, nout, n_dst_p, w), jnp.float32),
        mesh=_MESH,
        scratch_types=[
            pltpu.VMEM((epw2,), jnp.int32),
            pltpu.VMEM((epw2,), jnp.int32),
            pltpu.VMEM((ku, w), jnp.float32),
            pltpu.VMEM((ku, w), jnp.float32),
            pltpu.VMEM((zr, w), jnp.float32),
            pltpu.VMEM_SHARED((n_dst_p, w), jnp.float32),
            pltpu.SemaphoreType.DMA,
            pltpu.SemaphoreType.DMA,
            pltpu.SemaphoreType.DMA,
            pltpu.SemaphoreType.DMA,
        ],
        compiler_params=pltpu.CompilerParams(use_tc_tiling_on_sc=False),
    )


@functools.cache
def _make_label_gather():
    def body(zu, zm, iu_h, im_h, ou, om, iu, im, ba, bb, ga, gb, sa, sb):
        c = lax.axis_index("c")
        s = lax.axis_index("s")
        wid = c * NS + s
        pltpu.sync_copy(iu_h.at[c, s], iu)
        pltpu.sync_copy(im_h.at[c, s], im)
        base = wid * LPW

        def gath(tab, idx, blk, buf, sem):
            pltpu.async_copy(tab.at[idx.at[pl.ds(blk * KD, KD)]], buf, sem)

        def write(o, blk, buf, sem):
            pltpu.async_copy(buf, o.at[pl.ds(base + blk * KD, KD), :], sem)

        def drain(buf, sem):
            pltpu.make_async_copy(zu.at[pl.ds(0, KD)], buf, sem).wait()

        gath(zu, iu, 0, ba, ga)
        gath(zm, im, 0, bb, gb)

        def blkloop(h, _):
            drain(ba, ga)
            write(ou, h, ba, sa)
            drain(bb, gb)
            write(om, h, bb, sb)
            drain(ba, sa)
            gath(zu, iu, h + 1, ba, ga)
            drain(bb, sb)
            gath(zm, im, h + 1, bb, gb)
            return 0

        lax.fori_loop(0, DBLK, blkloop, 0)
        drain(ba, ga)
        drain(bb, gb)

    return pl.kernel(
        body,
        out_type=[
            jax.ShapeDtypeStruct((LPAD, 128), jnp.float32),
            jax.ShapeDtypeStruct((LPAD, 128), jnp.float32),
        ],
        mesh=_MESH,
        scratch_types=[
            pltpu.VMEM((LPW2,), jnp.int32),
            pltpu.VMEM((LPW2,), jnp.int32),
            pltpu.VMEM((KD, 128), jnp.float32),
            pltpu.VMEM((KD, 128), jnp.float32),
            pltpu.SemaphoreType.DMA,
            pltpu.SemaphoreType.DMA,
            pltpu.SemaphoreType.DMA,
            pltpu.SemaphoreType.DMA,
        ],
    )


_RB = 512


def _make_combine1_body(w, nch, wout):
    nco = 128 // wout

    def body(pref, xref, wl, wr, bl, href, hcref, invref):
        p = pref[...]
        agg = jnp.concatenate([p[0, cc] + p[1, cc] for cc in range(nch)],
                              axis=1)
        cnt = p[0, nch, :, 0:1] + p[1, nch, :, 0:1]
        inv = 1.0 / jnp.maximum(cnt, 1.0)
        h = (jnp.dot(agg * inv, wl[...], preferred_element_type=jnp.float32)
             + jnp.dot(xref[...], wr[...], preferred_element_type=jnp.float32)
             + bl[...])
        h = jnp.maximum(h, 0.0)
        href[...] = h
        invref[...] = inv
        for cc in range(nco):
            hcref[cc] = h[:, wout * cc:wout * (cc + 1)]
    return body


def _make_combine2_body(w, nch):
    def body(pref, xref, invref, wl, wr, bl, zref):
        p = pref[...]
        agg = jnp.concatenate([p[0, cc] + p[1, cc] for cc in range(nch)],
                              axis=1)
        zref[...] = (jnp.dot(agg * invref[...], wl[...],
                             preferred_element_type=jnp.float32)
                     + jnp.dot(xref[...], wr[...],
                               preferred_element_type=jnp.float32)
                     + bl[...])
    return body


def _wspec():
    return pl.BlockSpec((128, 128), lambda r: (0, 0))


def _bspec():
    return pl.BlockSpec((1, 128), lambda r: (0, 0))


def _combine1(P, x, Wl, Wr, bl, *, n, npad, w, nch, wout):
    grid = (npad + _RB - 1) // _RB
    nco = 128 // wout
    return pl.pallas_call(
        _make_combine1_body(w, nch, wout),
        grid=(grid,),
        in_specs=[
            pl.BlockSpec((NC, nch + 1, _RB, w), lambda r: (0, 0, r, 0)),
            pl.BlockSpec((_RB, 128), lambda r: (r, 0)),
            _wspec(), _wspec(), _bspec(),
        ],
        out_specs=[
            pl.BlockSpec((_RB, 128), lambda r: (r, 0)),
            pl.BlockSpec((nco, _RB, wout), lambda r: (0, r, 0)),
            pl.BlockSpec((_RB, 1), lambda r: (r, 0)),
        ],
        out_shape=[
            jax.ShapeDtypeStruct((n, 128), jnp.float32),
            jax.ShapeDtypeStruct((nco, n, wout), jnp.float32),
            jax.ShapeDtypeStruct((npad, 1), jnp.float32),
        ],
    )(P, x, Wl, Wr, bl)


def _combine2(P, x, inv, Wl, Wr, bl, *, n, npad, w, nch):
    grid = (npad + _RB - 1) // _RB
    return pl.pallas_call(
        _make_combine2_body(w, nch),
        grid=(grid,),
        in_specs=[
            pl.BlockSpec((NC, nch, _RB, w), lambda r: (0, 0, r, 0)),
            pl.BlockSpec((_RB, 128), lambda r: (r, 0)),
            pl.BlockSpec((_RB, 1), lambda r: (r, 0)),
            _wspec(), _wspec(), _bspec(),
        ],
        out_specs=pl.BlockSpec((_RB, 128), lambda r: (r, 0)),
        out_shape=jax.ShapeDtypeStruct((n, 128), jnp.float32),
    )(P, x, inv, Wl, Wr, bl)


def _decoder_body(zuref, zmref, w1a, w1b, b1, w2, b2, oref):
    h = (jnp.dot(zuref[...], w1a[...], preferred_element_type=jnp.float32)
         + jnp.dot(zmref[...], w1b[...], preferred_element_type=jnp.float32)
         + b1[...])
    h = jnp.maximum(h, 0.0)
    oref[...] = jnp.sum(h * w2[...], axis=1, keepdims=True) + b2[...]


def _decoder(zug, zmg, Wd1, bd1, Wd2, bd2):
    grid = LPAD // _RB
    return pl.pallas_call(
        _decoder_body,
        grid=(grid,),
        in_specs=[
            pl.BlockSpec((_RB, 128), lambda r: (r, 0)),
            pl.BlockSpec((_RB, 128), lambda r: (r, 0)),
            _wspec(), _wspec(), _bspec(), _bspec(),
            pl.BlockSpec((1, 1), lambda r: (0, 0)),
        ],
        out_specs=pl.BlockSpec((_RB, 1), lambda r: (r, 0)),
        out_shape=jax.ShapeDtypeStruct((LPAD, 1), jnp.float32),
    )(zug, zmg, Wd1[:128], Wd1[128:], bd1.reshape(1, 128),
      Wd2.reshape(1, 128), bd2.reshape(1, 1))


def _prep_edges(ix, padval, ku):
    epw2 = EPW + 2 * ku
    body = jnp.concatenate([ix, jnp.full((EPAD - E,), padval, jnp.int32)])
    body = body.reshape(NC, NS, EPW)
    tail = jnp.zeros((NC, NS, epw2 - EPW), jnp.int32)
    return jnp.concatenate([body, tail], axis=2)


def _prep_labels(ix):
    pad = jnp.zeros((LPAD - L,), jnp.int32)
    body = jnp.concatenate([ix, pad]).reshape(NC, NS, LPW)
    tail = jnp.zeros((NC, NS, LPW2 - LPW), jnp.int32)
    return jnp.concatenate([body, tail], axis=2)


def _chunks(x, w):
    nch = 128 // w
    return jnp.stack([x[:, w * cc:w * (cc + 1)] for cc in range(nch)])


def kernel(x_user, x_movie, edge_src, edge_dst, label_src, label_dst,
           Wl1u, bl1u, Wr1u, Wl1m, bl1m, Wr1m,
           Wl2u, bl2u, Wr2u, Wl2m, bl2m, Wr2m,
           Wd1, bd1, Wd2, bd2):
    eg_d_u = _prep_edges(edge_dst, 0, 1024)
    es_u = _prep_edges(edge_src, N_U, 1024)
    eg_s_m = _prep_edges(edge_src, 0, 1024)
    es_m = _prep_edges(edge_dst, N_M, 1024)

    xm_c = _chunks(x_movie, 8)
    xu_c = _chunks(x_user, 32)

    agg_u1 = _make_agg(N_M, NUP, 8, 16, 1024, True)
    agg_u2 = _make_agg(N_M, NUP, 8, 16, 1024, False)
    agg_m1 = _make_agg(N_U, NMP, 32, 4, 1024, True)
    agg_m2 = _make_agg(N_U, NMP, 32, 4, 1024, False)

    zo8 = jnp.stack([jnp.zeros((1024, 8), jnp.float32),
                     jnp.ones((1024, 8), jnp.float32)])
    zo32 = jnp.stack([jnp.zeros((512, 32), jnp.float32),
                      jnp.ones((512, 32), jnp.float32)])
    Pu1 = agg_u1(xm_c, eg_d_u, es_u, zo8)
    tok = (Pu1[0, 0, 0, 0] * 0.0).astype(jnp.int32)
    Pm1 = agg_m1(xu_c, eg_s_m, es_m + tok, zo32)

    h_u, hu_c, inv_u = _combine1(
        Pu1, x_user, Wl1u, Wr1u, bl1u.reshape(1, 128), n=N_U, npad=NUP,
        w=8, nch=16, wout=32)
    h_m, hm_c, inv_m = _combine1(
        Pm1, x_movie, Wl1m, Wr1m, bl1m.reshape(1, 128), n=N_M, npad=NMP,
        w=32, nch=4, wout=8)

    tok1 = (Pm1[0, 0, 0, 0] * 0.0).astype(jnp.int32)
    Pu2 = agg_u2(hm_c, eg_d_u, es_u + tok1, zo8)
    tok2 = (Pu2[0, 0, 0, 0] * 0.0).astype(jnp.int32)
    Pm2 = agg_m2(hu_c, eg_s_m, es_m + tok2, zo32)

    z_u = _combine2(Pu2, h_u, inv_u, Wl2u, Wr2u, bl2u.reshape(1, 128),
                    n=N_U, npad=NUP, w=8, nch=16)
    z_m = _combine2(Pm2, h_m, inv_m, Wl2m, Wr2m, bl2m.reshape(1, 128),
                    n=N_M, npad=NMP, w=32, nch=4)

    zug, zmg = _make_label_gather()(z_u, z_m, _prep_labels(label_src),
                                    _prep_labels(label_dst))
    o = _decoder(zug, zmg, Wd1, bd1, Wd2, bd2)
    return o[:L, 0]
